# baseline (speedup 1.0000x reference)
import functools

import jax
import jax.numpy as jnp
from jax import lax
from jax.experimental import pallas as pl
from jax.experimental.pallas import tpu as pltpu
from jax.experimental.pallas import tpu_sc as plsc

N = 10000
E = 320000
NC = 1
NS = 16
NW = NC * NS
CHUNK = 128
K = 327680 // (NW * CHUNK)
SLOTS = NW * K * CHUNK
TRASH = N
N_PAD = 10240
STRIPE = N_PAD // NS

_f32 = jnp.float32


def _mesh():
    return plsc.VectorSubcoreMesh(
        core_axis_name="c", subcore_axis_name="s",
        num_cores=NC, num_subcores=NS)


def _deg_body(dst_hbm, ones_hbm, zeros_hbm, out_hbm, dst_v, ones_v, stage_v, acc_sh):
    c = lax.axis_index("c")
    s = lax.axis_index("s")
    wid = c * NS + s
    pltpu.sync_copy(dst_hbm.at[wid], dst_v)
    pltpu.sync_copy(ones_hbm, ones_v)
    pltpu.sync_copy(zeros_hbm, stage_v)
    pltpu.sync_copy(stage_v, acc_sh.at[pl.ds(s * STRIPE, STRIPE)])
    plsc.subcore_barrier()

    def body(j, carry):
        pltpu.sync_copy(ones_v, acc_sh.at[dst_v.at[j]], add=True)
        return carry

    lax.fori_loop(0, K, body, 0)
    plsc.subcore_barrier()
    pltpu.sync_copy(acc_sh.at[pl.ds(s * STRIPE, STRIPE)], stage_v)
    pltpu.sync_copy(stage_v, out_hbm.at[c, pl.ds(s * STRIPE, STRIPE)])


def _deg_call(dst_p, ones_c, zeros_s):
    return pl.kernel(
        _deg_body,
        out_type=jax.ShapeDtypeStruct((NC, N_PAD), _f32),
        mesh=_mesh(),
        scratch_types=[
            pltpu.VMEM((K, CHUNK), jnp.int32),
            pltpu.VMEM((CHUNK,), _f32),
            pltpu.VMEM((STRIPE,), _f32),
            pltpu.VMEM_SHARED((N_PAD,), _f32),
        ],
        compiler_params=pltpu.CompilerParams(use_tc_tiling_on_sc=False),
    )(dst_p, ones_c, zeros_s)


NSLOT = 4
LEAD = 2


def _agg_body(u_hbm, src_hbm, dst_hbm, zeros_hbm, out_hbm,
              src_v, dst_v, rows_v, stage_v, acc_sh, gsem, ssem):
    c = lax.axis_index("c")
    s = lax.axis_index("s")
    wid = c * NS + s
    pltpu.sync_copy(src_hbm.at[wid], src_v)
    pltpu.sync_copy(dst_hbm.at[wid], dst_v)
    pltpu.sync_copy(zeros_hbm, stage_v)
    for t in range(STRIPE // CHUNK):
        pltpu.sync_copy(stage_v, acc_sh.at[pl.ds(s * STRIPE + t * CHUNK, CHUNK)])
    plsc.subcore_barrier()

    def gather(j, slot):
        pltpu.async_copy(u_hbm.at[src_v.at[j]], rows_v.at[slot], gsem.at[slot])

    def gather_wait(slot):
        pltpu.make_async_copy(u_hbm.at[pl.ds(0, CHUNK)], rows_v.at[slot],
                              gsem.at[slot]).wait()

    def scatter(j, slot):
        pltpu.async_copy(rows_v.at[slot], acc_sh.at[dst_v.at[j]],
                         ssem.at[slot], add=True)

    def scatter_wait(slot):
        pltpu.make_async_copy(u_hbm.at[pl.ds(0, CHUNK)], rows_v.at[slot],
                              ssem.at[slot]).wait()

    for b in range(LEAD):
        gather(b, b)

    def body(m, carry):
        for b in range(NSLOT):
            j = m * NSLOT + b
            slot = b
            gather_wait(slot)
            scatter(j, slot)
            nslot = (b + LEAD) % NSLOT
            nj = j + LEAD

            @pl.when(nj < K)
            def _():
                @pl.when(nj >= NSLOT)
                def _():
                    scatter_wait(nslot)
                gather(nj, nslot)
        return carry

    lax.fori_loop(0, K // NSLOT, body, 0)
    for b in range(NSLOT):
        scatter_wait(b)
    plsc.subcore_barrier()
    for t in range(STRIPE // CHUNK):
        pltpu.sync_copy(acc_sh.at[pl.ds(s * STRIPE + t * CHUNK, CHUNK)], stage_v)
        pltpu.sync_copy(stage_v, out_hbm.at[c, pl.ds(s * STRIPE + t * CHUNK, CHUNK)])


def _agg_call(u_pad, src_p, dst_p, zeros_sh, h):
    return pl.kernel(
        _agg_body,
        out_type=jax.ShapeDtypeStruct((NC, N_PAD, h), _f32),
        mesh=_mesh(),
        scratch_types=[
            pltpu.VMEM((K, CHUNK), jnp.int32),
            pltpu.VMEM((K, CHUNK), jnp.int32),
            pltpu.VMEM((NSLOT, CHUNK, h), _f32),
            pltpu.VMEM((CHUNK, h), _f32),
            pltpu.VMEM_SHARED((N_PAD, h), _f32),
            pltpu.SemaphoreType.DMA((NSLOT,)),
            pltpu.SemaphoreType.DMA((NSLOT,)),
        ],
        compiler_params=pltpu.CompilerParams(use_tc_tiling_on_sc=False),
    )(u_pad, src_p, dst_p, zeros_sh)


def _tc_pre_body(x_ref, w1_ref, dp_ref, u1_ref, dinv_ref):
    deg = sum(dp_ref[i] for i in range(NC)) + 1.0
    rowmask = lax.broadcasted_iota(jnp.int32, (N_PAD, 1), 0) < N
    dinv = jnp.where(rowmask, lax.rsqrt(deg), 0.0)
    u1 = jnp.dot(x_ref[...], w1_ref[...], preferred_element_type=_f32) * dinv
    u1_ref[...] = u1
    dinv_ref[...] = dinv


def _tc_mid_body(sp_ref, u_ref, dinv_ref, b_ref, g_ref, be_ref, w_ref, out_ref):
    dinv = dinv_ref[...]
    stot = sum(sp_ref[i] for i in range(NC)) + u_ref[...]
    cvt = stot * dinv + b_ref[...]
    rowmask = (lax.broadcasted_iota(jnp.int32, (N_PAD, 1), 0) < N).astype(_f32)
    mean = jnp.sum(cvt * rowmask, axis=0, keepdims=True) * (1.0 / N)
    dctr = (cvt - mean) * rowmask
    var = jnp.sum(dctr * dctr, axis=0, keepdims=True) * (1.0 / N)
    y = jnp.maximum((cvt - mean) * lax.rsqrt(var + 1e-5) * g_ref[...] + be_ref[...], 0.0)
    out_ref[...] = jnp.dot(y, w_ref[...], preferred_element_type=_f32) * dinv


def _tc_fin_body(sp_ref, u_ref, dinv_ref, b3_ref, w4_ref, b4_ref, out_ref):
    stot = sum(sp_ref[i] for i in range(NC)) + u_ref[...]
    y = jnp.maximum(stot * dinv_ref[...] + b3_ref[...], 0.0)
    out_ref[...] = jnp.dot(y, w4_ref[...], preferred_element_type=_f32) + b4_ref[...]


def _tc_pre(x_p, W1, dp):
    return pl.pallas_call(
        _tc_pre_body,
        out_shape=(jax.ShapeDtypeStruct((N_PAD, 64), _f32),
                   jax.ShapeDtypeStruct((N_PAD, 1), _f32)),
    )(x_p, W1, dp)


def _tc_mid(sp, u, dinv, b, g, be, W, h2):
    return pl.pallas_call(
        _tc_mid_body,
        out_shape=jax.ShapeDtypeStruct((N_PAD, h2), _f32),
    )(sp, u, dinv, b, g, be, W)


def _tc_fin(sp, u, dinv, b3, W4, b4):
    return pl.pallas_call(
        _tc_fin_body,
        out_shape=jax.ShapeDtypeStruct((N_PAD, 1), _f32),
    )(sp, u, dinv, b3, W4, b4)


def kernel(x, edge_index, W1, b1, g1, be1, W2, b2, g2, be2, W3, b3, W4, b4):
    src = edge_index[0]
    dst = edge_index[1]
    pad = SLOTS - E
    pad_dst = TRASH + (jnp.arange(pad, dtype=dst.dtype) % (N_PAD - N))
    src_p = jnp.concatenate(
        [src, jnp.full((pad,), TRASH, src.dtype)]).reshape(NW, K, CHUNK)
    dst_p = jnp.concatenate([dst, pad_dst]).reshape(NW, K, CHUNK)
    x_p = jnp.pad(x, ((0, N_PAD - N), (0, 0)))

    ones_c = jnp.ones((CHUNK,), _f32)
    zeros_s = jnp.zeros((STRIPE,), _f32)
    zeros64 = jnp.zeros((CHUNK, 64), _f32)
    zeros32 = jnp.zeros((CHUNK, 32), _f32)

    deg_part = _deg_call(dst_p, ones_c, zeros_s)
    u1, dinv = _tc_pre(x_p, W1, deg_part.reshape(NC, N_PAD, 1))
    s1 = _agg_call(u1, src_p, dst_p, zeros64, 64)
    u2 = _tc_mid(s1, u1, dinv, b1.reshape(1, -1), g1.reshape(1, -1),
                 be1.reshape(1, -1), W2, 64)
    s2 = _agg_call(u2, src_p, dst_p, zeros64, 64)
    u3 = _tc_mid(s2, u2, dinv, b2.reshape(1, -1), g2.reshape(1, -1),
                 be2.reshape(1, -1), W3, 32)
    s3 = _agg_call(u3, src_p, dst_p, zeros32, 32)
    out = _tc_fin(s3, u3, dinv, b3.reshape(1, -1), W4, b4.reshape(1, 1))
    return out[:N]


# speedup vs baseline: 2.8486x; 2.8486x over previous
import functools

import jax
import jax.numpy as jnp
from jax import lax
from jax.experimental import pallas as pl
from jax.experimental.pallas import tpu as pltpu
from jax.experimental.pallas import tpu_sc as plsc

N = 10000
E = 320000
NC = 2
NS = 16
NW = NC * NS
CHUNK = 128
K = 327680 // (NW * CHUNK)
SLOTS = NW * K * CHUNK
TRASH = N
N_PAD = 10240
STRIPE = N_PAD // NS

_f32 = jnp.float32


def _mesh():
    return plsc.VectorSubcoreMesh(
        core_axis_name="c", subcore_axis_name="s",
        num_cores=NC, num_subcores=NS)


def _deg_body(dst_hbm, ones_hbm, zeros_hbm, out_hbm, dst_v, ones_v, stage_v, acc_sh):
    c = lax.axis_index("c")
    s = lax.axis_index("s")
    wid = c * NS + s
    pltpu.sync_copy(dst_hbm.at[wid], dst_v)
    pltpu.sync_copy(ones_hbm, ones_v)
    pltpu.sync_copy(zeros_hbm, stage_v)
    pltpu.sync_copy(stage_v, acc_sh.at[pl.ds(s * STRIPE, STRIPE)])
    plsc.subcore_barrier()

    def body(j, carry):
        pltpu.sync_copy(ones_v, acc_sh.at[dst_v.at[j]], add=True)
        return carry

    lax.fori_loop(0, K, body, 0)
    plsc.subcore_barrier()
    pltpu.sync_copy(acc_sh.at[pl.ds(s * STRIPE, STRIPE)], stage_v)
    pltpu.sync_copy(stage_v, out_hbm.at[c, pl.ds(s * STRIPE, STRIPE)])


def _deg_call(dst_p, ones_c, zeros_s):
    return pl.kernel(
        _deg_body,
        out_type=jax.ShapeDtypeStruct((NC, N_PAD), _f32),
        mesh=_mesh(),
        scratch_types=[
            pltpu.VMEM((K, CHUNK), jnp.int32),
            pltpu.VMEM((CHUNK,), _f32),
            pltpu.VMEM((STRIPE,), _f32),
            pltpu.VMEM_SHARED((N_PAD,), _f32),
        ],
        compiler_params=pltpu.CompilerParams(use_tc_tiling_on_sc=False),
    )(dst_p, ones_c, zeros_s)


NSLOT = 4
LEAD = 2


def _agg_body(u_hbm, src_hbm, dst_hbm, zeros_hbm, out_hbm,
              src_v, dst_v, rows_v, stage_v, acc_sh, gsem, ssem):
    c = lax.axis_index("c")
    s = lax.axis_index("s")
    wid = c * NS + s
    pltpu.sync_copy(src_hbm.at[wid], src_v)
    pltpu.sync_copy(dst_hbm.at[wid], dst_v)
    pltpu.sync_copy(zeros_hbm, stage_v)
    for t in range(STRIPE // CHUNK):
        pltpu.sync_copy(stage_v, acc_sh.at[pl.ds(s * STRIPE + t * CHUNK, CHUNK)])
    plsc.subcore_barrier()

    def gather(j, slot):
        pltpu.async_copy(u_hbm.at[src_v.at[j]], rows_v.at[slot], gsem.at[slot])

    def gather_wait(slot):
        pltpu.make_async_copy(u_hbm.at[pl.ds(0, CHUNK)], rows_v.at[slot],
                              gsem.at[slot]).wait()

    def scatter(j, slot):
        pltpu.async_copy(rows_v.at[slot], acc_sh.at[dst_v.at[j]],
                         ssem.at[slot], add=True)

    def scatter_wait(slot):
        pltpu.make_async_copy(u_hbm.at[pl.ds(0, CHUNK)], rows_v.at[slot],
                              ssem.at[slot]).wait()

    for b in range(LEAD):
        gather(b, b)

    def body(m, carry):
        for b in range(NSLOT):
            j = m * NSLOT + b
            slot = b
            gather_wait(slot)
            scatter(j, slot)
            nslot = (b + LEAD) % NSLOT
            nj = j + LEAD

            @pl.when(nj < K)
            def _():
                @pl.when(nj >= NSLOT)
                def _():
                    scatter_wait(nslot)
                gather(nj, nslot)
        return carry

    lax.fori_loop(0, K // NSLOT, body, 0)
    for b in range(NSLOT):
        scatter_wait(b)
    plsc.subcore_barrier()
    for t in range(STRIPE // CHUNK):
        pltpu.sync_copy(acc_sh.at[pl.ds(s * STRIPE + t * CHUNK, CHUNK)], stage_v)
        pltpu.sync_copy(stage_v, out_hbm.at[c, pl.ds(s * STRIPE + t * CHUNK, CHUNK)])


def _agg_call(u_pad, src_p, dst_p, zeros_sh, h):
    return pl.kernel(
        _agg_body,
        out_type=jax.ShapeDtypeStruct((NC, N_PAD, h), _f32),
        mesh=_mesh(),
        scratch_types=[
            pltpu.VMEM((K, CHUNK), jnp.int32),
            pltpu.VMEM((K, CHUNK), jnp.int32),
            pltpu.VMEM((NSLOT, CHUNK, h), _f32),
            pltpu.VMEM((CHUNK, h), _f32),
            pltpu.VMEM_SHARED((N_PAD, h), _f32),
            pltpu.SemaphoreType.DMA((NSLOT,)),
            pltpu.SemaphoreType.DMA((NSLOT,)),
        ],
        compiler_params=pltpu.CompilerParams(use_tc_tiling_on_sc=False),
    )(u_pad, src_p, dst_p, zeros_sh)


def _tc_pre_body(x_ref, w1_ref, dp_ref, u1_ref, dinv_ref):
    deg = sum(dp_ref[i] for i in range(NC)) + 1.0
    rowmask = lax.broadcasted_iota(jnp.int32, (N_PAD, 1), 0) < N
    dinv = jnp.where(rowmask, lax.rsqrt(deg), 0.0)
    u1 = jnp.dot(x_ref[...], w1_ref[...], preferred_element_type=_f32) * dinv
    u1_ref[...] = u1
    dinv_ref[...] = dinv


def _tc_mid_body(sp_ref, u_ref, dinv_ref, b_ref, g_ref, be_ref, w_ref, out_ref):
    dinv = dinv_ref[...]
    stot = sum(sp_ref[i] for i in range(NC)) + u_ref[...]
    cvt = stot * dinv + b_ref[...]
    rowmask = (lax.broadcasted_iota(jnp.int32, (N_PAD, 1), 0) < N).astype(_f32)
    mean = jnp.sum(cvt * rowmask, axis=0, keepdims=True) * (1.0 / N)
    dctr = (cvt - mean) * rowmask
    var = jnp.sum(dctr * dctr, axis=0, keepdims=True) * (1.0 / N)
    y = jnp.maximum((cvt - mean) * lax.rsqrt(var + 1e-5) * g_ref[...] + be_ref[...], 0.0)
    out_ref[...] = jnp.dot(y, w_ref[...], preferred_element_type=_f32) * dinv


def _tc_fin_body(sp_ref, u_ref, dinv_ref, b3_ref, w4_ref, b4_ref, out_ref):
    stot = sum(sp_ref[i] for i in range(NC)) + u_ref[...]
    y = jnp.maximum(stot * dinv_ref[...] + b3_ref[...], 0.0)
    out_ref[...] = jnp.dot(y, w4_ref[...], preferred_element_type=_f32) + b4_ref[...]


def _tc_pre(x_p, W1, dp):
    return pl.pallas_call(
        _tc_pre_body,
        out_shape=(jax.ShapeDtypeStruct((N_PAD, 64), _f32),
                   jax.ShapeDtypeStruct((N_PAD, 1), _f32)),
    )(x_p, W1, dp)


def _tc_mid(sp, u, dinv, b, g, be, W, h2):
    return pl.pallas_call(
        _tc_mid_body,
        out_shape=jax.ShapeDtypeStruct((N_PAD, h2), _f32),
    )(sp, u, dinv, b, g, be, W)


def _tc_fin(sp, u, dinv, b3, W4, b4):
    return pl.pallas_call(
        _tc_fin_body,
        out_shape=jax.ShapeDtypeStruct((N_PAD, 1), _f32),
    )(sp, u, dinv, b3, W4, b4)


def kernel(x, edge_index, W1, b1, g1, be1, W2, b2, g2, be2, W3, b3, W4, b4):
    src = edge_index[0]
    dst = edge_index[1]
    pad = SLOTS - E
    pad_src = jnp.arange(pad, dtype=src.dtype) % N
    pad_dst = TRASH + (jnp.arange(pad, dtype=dst.dtype) % (N_PAD - N))
    src_p = jnp.concatenate([src, pad_src]).reshape(NW, K, CHUNK)
    dst_p = jnp.concatenate([dst, pad_dst]).reshape(NW, K, CHUNK)
    x_p = jnp.pad(x, ((0, N_PAD - N), (0, 0)))

    ones_c = jnp.ones((CHUNK,), _f32)
    zeros_s = jnp.zeros((STRIPE,), _f32)
    zeros64 = jnp.zeros((CHUNK, 64), _f32)
    zeros32 = jnp.zeros((CHUNK, 32), _f32)

    deg_part = _deg_call(dst_p, ones_c, zeros_s)
    u1, dinv = _tc_pre(x_p, W1, deg_part.reshape(NC, N_PAD, 1))
    s1 = _agg_call(u1, src_p, dst_p, zeros64, 64)
    u2 = _tc_mid(s1, u1, dinv, b1.reshape(1, -1), g1.reshape(1, -1),
                 be1.reshape(1, -1), W2, 64)
    s2 = _agg_call(u2, src_p, dst_p, zeros64, 64)
    u3 = _tc_mid(s2, u2, dinv, b2.reshape(1, -1), g2.reshape(1, -1),
                 be2.reshape(1, -1), W3, 32)
    s3 = _agg_call(u3, src_p, dst_p, zeros32, 32)
    out = _tc_fin(s3, u3, dinv, b3.reshape(1, -1), W4, b4.reshape(1, 1))
    return out[:N]


# speedup vs baseline: 3.3118x; 1.1626x over previous
import jax
import jax.numpy as jnp
import numpy as np
from jax import lax
from jax.experimental import pallas as pl
from jax.experimental.pallas import tpu as pltpu
from jax.experimental.pallas import tpu_sc as plsc

N = 10000
E = 320000
NC = 2
NS = 16
NW = NC * NS
CHUNK = 128
K = 327680 // (NW * CHUNK)
SLOTS = NW * K * CHUNK
TRASH = N
N_PAD = 10240
STRIPE = N_PAD // NS

_f32 = jnp.float32


def _mesh():
    return plsc.VectorSubcoreMesh(
        core_axis_name="c", subcore_axis_name="s",
        num_cores=NC, num_subcores=NS)


def _deg_body(dst_hbm, ones_hbm, zeros_hbm, out_hbm, dst_v, ones_v, stage_v, acc_sh, dsem):
    c = lax.axis_index("c")
    s = lax.axis_index("s")
    wid = c * NS + s
    pltpu.sync_copy(dst_hbm.at[wid], dst_v)
    pltpu.sync_copy(ones_hbm, ones_v)
    pltpu.sync_copy(zeros_hbm, stage_v)
    pltpu.sync_copy(stage_v, acc_sh.at[pl.ds(s * STRIPE, STRIPE)])
    plsc.subcore_barrier()

    def body(j, carry):
        pltpu.async_copy(ones_v, acc_sh.at[dst_v.at[j]], dsem, add=True)
        return carry

    lax.fori_loop(0, K, body, 0)

    def drain(j, carry):
        pltpu.make_async_copy(ones_hbm, ones_v, dsem).wait()
        return carry

    lax.fori_loop(0, K, drain, 0)
    plsc.subcore_barrier()
    pltpu.sync_copy(acc_sh.at[pl.ds(s * STRIPE, STRIPE)], stage_v)
    pltpu.sync_copy(stage_v, out_hbm.at[c, pl.ds(s * STRIPE, STRIPE)])


def _deg_call(dst_p, ones_c, zeros_s):
    return pl.kernel(
        _deg_body,
        out_type=jax.ShapeDtypeStruct((NC, N_PAD), _f32),
        mesh=_mesh(),
        scratch_types=[
            pltpu.VMEM((K, CHUNK), jnp.int32),
            pltpu.VMEM((CHUNK,), _f32),
            pltpu.VMEM((STRIPE,), _f32),
            pltpu.VMEM_SHARED((N_PAD,), _f32),
            pltpu.SemaphoreType.DMA,
        ],
        compiler_params=pltpu.CompilerParams(use_tc_tiling_on_sc=False),
    )(dst_p, ones_c, zeros_s)


NSLOT = 5
LEAD = 3
WCH = STRIPE // CHUNK


def _agg_body(u_hbm, src_hbm, dst_hbm, zeros_hbm, out_hbm,
              src_v, dst_v, rows_v, stage_v, acc_sh, gsem, ssem, wsem):
    c = lax.axis_index("c")
    s = lax.axis_index("s")
    wid = c * NS + s
    pltpu.sync_copy(src_hbm.at[wid], src_v)
    pltpu.sync_copy(dst_hbm.at[wid], dst_v)
    pltpu.sync_copy(zeros_hbm, stage_v)
    for t in range(WCH):
        pltpu.async_copy(stage_v, acc_sh.at[pl.ds(s * STRIPE + t * CHUNK, CHUNK)],
                         wsem.at[t])
    for t in range(WCH):
        pltpu.make_async_copy(zeros_hbm, stage_v, wsem.at[t]).wait()
    plsc.subcore_barrier()

    def gather(j, slot):
        pltpu.async_copy(u_hbm.at[src_v.at[j]], rows_v.at[slot], gsem.at[slot])

    def gather_wait(slot):
        pltpu.make_async_copy(u_hbm.at[pl.ds(0, CHUNK)], rows_v.at[slot],
                              gsem.at[slot]).wait()

    def scatter(j, slot):
        pltpu.async_copy(rows_v.at[slot], acc_sh.at[dst_v.at[j]],
                         ssem.at[slot], add=True)

    def scatter_wait(slot):
        pltpu.make_async_copy(u_hbm.at[pl.ds(0, CHUNK)], rows_v.at[slot],
                              ssem.at[slot]).wait()

    for b in range(LEAD):
        gather(b, b)

    def body(m, carry):
        for b in range(NSLOT):
            j = m * NSLOT + b
            slot = b
            gather_wait(slot)
            scatter(j, slot)
            nslot = (b + LEAD) % NSLOT
            nj = j + LEAD

            @pl.when(nj < K)
            def _():
                @pl.when(nj >= NSLOT)
                def _():
                    scatter_wait(nslot)
                gather(nj, nslot)
        return carry

    lax.fori_loop(0, K // NSLOT, body, 0)
    for b in range(NSLOT):
        scatter_wait(b)
    plsc.subcore_barrier()
    bufs = [rows_v.at[t] for t in range(NSLOT)] + [stage_v]
    for t in range(WCH):
        pltpu.async_copy(acc_sh.at[pl.ds(s * STRIPE + t * CHUNK, CHUNK)],
                         bufs[t], wsem.at[t])
    for t in range(WCH):
        pltpu.make_async_copy(zeros_hbm, bufs[t], wsem.at[t]).wait()
        pltpu.async_copy(bufs[t], out_hbm.at[c, pl.ds(s * STRIPE + t * CHUNK, CHUNK)],
                         wsem.at[WCH + t])
    for t in range(WCH):
        pltpu.make_async_copy(zeros_hbm, bufs[t], wsem.at[WCH + t]).wait()


def _agg_call(u_pad, src_p, dst_p, zeros_sh, h):
    return pl.kernel(
        _agg_body,
        out_type=jax.ShapeDtypeStruct((NC, N_PAD, h), _f32),
        mesh=_mesh(),
        scratch_types=[
            pltpu.VMEM((K, CHUNK), jnp.int32),
            pltpu.VMEM((K, CHUNK), jnp.int32),
            pltpu.VMEM((NSLOT, CHUNK, h), _f32),
            pltpu.VMEM((CHUNK, h), _f32),
            pltpu.VMEM_SHARED((N_PAD, h), _f32),
            pltpu.SemaphoreType.DMA((NSLOT,)),
            pltpu.SemaphoreType.DMA((NSLOT,)),
            pltpu.SemaphoreType.DMA((2 * WCH,)),
        ],
        compiler_params=pltpu.CompilerParams(use_tc_tiling_on_sc=False),
    )(u_pad, src_p, dst_p, zeros_sh)


def _tc_pre_body(x_ref, w1_ref, dp_ref, u1_ref, dinv_ref):
    deg = sum(dp_ref[:, i:i + 1] for i in range(NC)) + 1.0
    rowmask = lax.broadcasted_iota(jnp.int32, (N_PAD, 1), 0) < N
    dinv = jnp.where(rowmask, lax.rsqrt(deg), 0.0)
    z = jnp.dot(x_ref[...], w1_ref[...], preferred_element_type=_f32)
    u1_ref[0:N, :] = z * dinv[0:N]
    u1_ref[N:N_PAD, :] = jnp.zeros((N_PAD - N, 64), _f32)
    dinv_ref[...] = dinv


def _tc_mid_body(sp_ref, u_ref, dinv_ref, b_ref, g_ref, be_ref, w_ref, out_ref):
    dinv = dinv_ref[...]
    stot = sum(sp_ref[i] for i in range(NC)) + u_ref[...]
    cvt = stot * dinv + b_ref[...]
    rowmask = (lax.broadcasted_iota(jnp.int32, (N_PAD, 1), 0) < N).astype(_f32)
    mean = jnp.sum(cvt * rowmask, axis=0, keepdims=True) * (1.0 / N)
    dctr = (cvt - mean) * rowmask
    var = jnp.sum(dctr * dctr, axis=0, keepdims=True) * (1.0 / N)
    y = jnp.maximum((cvt - mean) * lax.rsqrt(var + 1e-5) * g_ref[...] + be_ref[...], 0.0)
    out_ref[...] = jnp.dot(y, w_ref[...], preferred_element_type=_f32) * dinv


def _tc_fin_body(sp_ref, u_ref, dinv_ref, b3_ref, w4_ref, b4_ref, out_ref):
    stot = sum(sp_ref[i] for i in range(NC)) + u_ref[...]
    y = jnp.maximum(stot * dinv_ref[...] + b3_ref[...], 0.0)
    out_ref[...] = jnp.dot(y, w4_ref[...], preferred_element_type=_f32) + b4_ref[...]


def _tc_pre(x, W1, dp):
    return pl.pallas_call(
        _tc_pre_body,
        out_shape=(jax.ShapeDtypeStruct((N_PAD, 64), _f32),
                   jax.ShapeDtypeStruct((N_PAD, 1), _f32)),
    )(x, W1, dp)


def _tc_mid(sp, u, dinv, b, g, be, W, h2):
    return pl.pallas_call(
        _tc_mid_body,
        out_shape=jax.ShapeDtypeStruct((N_PAD, h2), _f32),
    )(sp, u, dinv, b, g, be, W)


def _tc_fin(sp, u, dinv, b3, W4, b4):
    return pl.pallas_call(
        _tc_fin_body,
        out_shape=jax.ShapeDtypeStruct((N_PAD, 1), _f32),
    )(sp, u, dinv, b3, W4, b4)


def kernel(x, edge_index, W1, b1, g1, be1, W2, b2, g2, be2, W3, b3, W4, b4):
    src = edge_index[0]
    dst = edge_index[1]
    pad = SLOTS - E
    pad_src = jnp.asarray(np.arange(pad, dtype=np.int32) % N)
    pad_dst = jnp.asarray(TRASH + (np.arange(pad, dtype=np.int32) % (N_PAD - N)))
    src_p = jnp.concatenate([src, pad_src]).reshape(NW, K, CHUNK)
    dst_p = jnp.concatenate([dst, pad_dst]).reshape(NW, K, CHUNK)

    ones_c = jnp.ones((CHUNK,), _f32)
    zeros_s = jnp.zeros((STRIPE,), _f32)
    zeros64 = jnp.zeros((CHUNK, 64), _f32)
    zeros32 = jnp.zeros((CHUNK, 32), _f32)

    deg_part = _deg_call(dst_p, ones_c, zeros_s)
    u1, dinv = _tc_pre(x, W1, jnp.transpose(deg_part))
    s1 = _agg_call(u1, src_p, dst_p, zeros64, 64)
    u2 = _tc_mid(s1, u1, dinv, b1.reshape(1, -1), g1.reshape(1, -1),
                 be1.reshape(1, -1), W2, 64)
    s2 = _agg_call(u2, src_p, dst_p, zeros64, 64)
    u3 = _tc_mid(s2, u2, dinv, b2.reshape(1, -1), g2.reshape(1, -1),
                 be2.reshape(1, -1), W3, 32)
    s3 = _agg_call(u3, src_p, dst_p, zeros32, 32)
    out = _tc_fin(s3, u3, dinv, b3.reshape(1, -1), W4, b4.reshape(1, 1))
    return out[:N]
